# baseline (speedup 1.0000x reference)
import jax
import jax.numpy as jnp
from jax.experimental import pallas as pl
from jax.experimental.pallas import tpu as pltpu

_SCALE = 30.0
_R2 = 0.7071067811865476
_LOG2E = 1.4426950408889634
_A = _SCALE * _LOG2E


def _psi(c):
    c = jnp.clip(c, -1.0, 1.0)
    c2 = c * c
    cos4 = 8.0 * c2 * c2 - 8.0 * c2 + 1.0
    k = (
        (c <= _R2).astype(jnp.int32)
        + (c <= 0.0).astype(jnp.int32)
        + (c <= -_R2).astype(jnp.int32)
    )
    co = jnp.where((k & 1) == 1, -1.0, 1.0)
    return co * cos4 - 2.0 * k.astype(jnp.float32)


def _sub_loss(yh, yv):
    cols = jax.lax.broadcasted_iota(jnp.int32, yh.shape, 1)
    mask = cols == yv
    w = jnp.where(mask, yh, 0.0)
    z = jnp.exp2(yh * _A)
    ones = jnp.ones((yh.shape[1], 1), dtype=jnp.float32)
    c = jax.lax.dot(w, ones, precision=jax.lax.Precision.HIGHEST)
    s0 = jax.lax.dot(z, ones, precision=jax.lax.Precision.HIGHEST)
    psi = _psi(c)
    s = s0 - jnp.exp2(c * _A) + jnp.exp2(psi * _A)
    lse = jnp.log(s)
    return jnp.sum(lse - _SCALE * psi)


def _body(a_ref, b_ref, c_ref, d_ref, ya_ref, yb_ref, yc_ref, yd_ref, out_ref):
    i = pl.program_id(0)
    nsteps = pl.num_programs(0)

    part = (
        _sub_loss(a_ref[...], ya_ref[...])
        + _sub_loss(b_ref[...], yb_ref[...])
        + _sub_loss(c_ref[...], yc_ref[...])
        + _sub_loss(d_ref[...], yd_ref[...])
    )

    @pl.when(i == 0)
    def _init():
        out_ref[0, 0] = 0.0

    out_ref[0, 0] += part

    @pl.when(i == nsteps - 1)
    def _final():
        out_ref[0, 0] = out_ref[0, 0] * (1.0 / (nsteps * 4 * a_ref.shape[0]))


def kernel(y_hat, y):
    n, num_class = y_hat.shape
    blk = 512
    grid = n // (4 * blk)
    y2 = y.reshape(n, 1)

    def mk(q):
        return pl.BlockSpec((blk, num_class), lambda i, q=q: (4 * i + q, 0))

    def mky(q):
        return pl.BlockSpec((blk, 1), lambda i, q=q: (4 * i + q, 0))

    out = pl.pallas_call(
        _body,
        grid=(grid,),
        in_specs=[mk(0), mk(1), mk(2), mk(3), mky(0), mky(1), mky(2), mky(3)],
        out_specs=pl.BlockSpec((1, 1), lambda i: (0, 0), memory_space=pltpu.SMEM),
        out_shape=jax.ShapeDtypeStruct((1, 1), jnp.float32),
    )(y_hat, y_hat, y_hat, y_hat, y2, y2, y2, y2)
    return out[0, 0]


# speedup vs baseline: 1.9234x; 1.9234x over previous
import jax
import jax.numpy as jnp
from jax.experimental import pallas as pl
from jax.experimental.pallas import tpu as pltpu

_SCALE = 30.0
_R2 = 0.7071067811865476
_LOG2E = 1.4426950408889634
_A = _SCALE * _LOG2E


def _psi(c):
    c = jnp.clip(c, -1.0, 1.0)
    c2 = c * c
    cos4 = 8.0 * c2 * c2 - 8.0 * c2 + 1.0
    k = (
        (c <= _R2).astype(jnp.int32)
        + (c <= 0.0).astype(jnp.int32)
        + (c <= -_R2).astype(jnp.int32)
    )
    co = jnp.where((k & 1) == 1, -1.0, 1.0)
    return co * cos4 - 2.0 * k.astype(jnp.float32)


def _sub_loss(yh, yv):
    cols = jax.lax.broadcasted_iota(jnp.int32, yh.shape, 1)
    mask = cols == yv
    c = jnp.sum(jnp.where(mask, yh, 0.0), axis=1, keepdims=True)
    psi = _psi(c)
    s0 = jnp.sum(jnp.exp2(yh * _A), axis=1, keepdims=True)
    s = s0 - jnp.exp2(c * _A) + jnp.exp2(psi * _A)
    lse = jnp.log(s)
    return jnp.sum(lse - _SCALE * psi)


def _body(*refs):
    out_ref = refs[-1]
    yh_refs = refs[:8]
    y_refs = refs[8:16]
    i = pl.program_id(0)
    nsteps = pl.num_programs(0)

    part = _sub_loss(yh_refs[0][...], y_refs[0][...])
    for q in range(1, 8):
        part = part + _sub_loss(yh_refs[q][...], y_refs[q][...])

    @pl.when(i == 0)
    def _init():
        out_ref[0, 0] = 0.0

    out_ref[0, 0] += part

    @pl.when(i == nsteps - 1)
    def _final():
        out_ref[0, 0] = out_ref[0, 0] * (1.0 / (nsteps * 8 * refs[0].shape[0]))


def kernel(y_hat, y):
    n, num_class = y_hat.shape
    blk = 256
    grid = n // (8 * blk)
    y2 = y.reshape(n, 1)

    def mk(q):
        return pl.BlockSpec((blk, num_class), lambda i, q=q: (8 * i + q, 0))

    def mky(q):
        return pl.BlockSpec((blk, 1), lambda i, q=q: (8 * i + q, 0))

    out = pl.pallas_call(
        _body,
        grid=(grid,),
        in_specs=[mk(q) for q in range(8)] + [mky(q) for q in range(8)],
        out_specs=pl.BlockSpec((1, 1), lambda i: (0, 0), memory_space=pltpu.SMEM),
        out_shape=jax.ShapeDtypeStruct((1, 1), jnp.float32),
    )(*([y_hat] * 8 + [y2] * 8))
    return out[0, 0]


# speedup vs baseline: 1.9240x; 1.0004x over previous
import jax
import jax.numpy as jnp
from jax.experimental import pallas as pl
from jax.experimental.pallas import tpu as pltpu

_SCALE = 30.0
_R2 = 0.7071067811865476
_LOG2E = 1.4426950408889634
_A = _SCALE * _LOG2E
NV = 2
BLK = 2048


def _psi(c):
    c = jnp.clip(c, -1.0, 1.0)
    c2 = c * c
    cos4 = 8.0 * c2 * c2 - 8.0 * c2 + 1.0
    k = (
        (c <= _R2).astype(jnp.int32)
        + (c <= 0.0).astype(jnp.int32)
        + (c <= -_R2).astype(jnp.int32)
    )
    co = jnp.where((k & 1) == 1, -1.0, 1.0)
    return co * cos4 - 2.0 * k.astype(jnp.float32)


def _sub_loss(yh, yv):
    cols = jax.lax.broadcasted_iota(jnp.int32, yh.shape, 1)
    mask = cols == yv
    c = jnp.sum(jnp.where(mask, yh, 0.0), axis=1, keepdims=True)
    psi = _psi(c)
    s0 = jnp.sum(jnp.exp2(yh * _A), axis=1, keepdims=True)
    s = s0 - jnp.exp2(c * _A) + jnp.exp2(psi * _A)
    lse = jnp.log(s)
    return jnp.sum(lse - _SCALE * psi)


def _body(*refs):
    out_ref = refs[-1]
    yh_refs = refs[:NV]
    y_refs = refs[NV:2 * NV]
    i = pl.program_id(0)
    nsteps = pl.num_programs(0)

    part = _sub_loss(yh_refs[0][...], y_refs[0][...])
    for q in range(1, NV):
        part = part + _sub_loss(yh_refs[q][...], y_refs[q][...])

    @pl.when(i == 0)
    def _init():
        out_ref[0, 0] = 0.0

    out_ref[0, 0] += part

    @pl.when(i == nsteps - 1)
    def _final():
        out_ref[0, 0] = out_ref[0, 0] * (1.0 / (nsteps * NV * refs[0].shape[0]))


def kernel(y_hat, y):
    n, num_class = y_hat.shape
    blk = BLK
    grid = n // (NV * blk)
    y2 = y.reshape(n, 1)

    def mk(q):
        return pl.BlockSpec((blk, num_class), lambda i, q=q: (NV * i + q, 0))

    def mky(q):
        return pl.BlockSpec((blk, 1), lambda i, q=q: (NV * i + q, 0))

    out = pl.pallas_call(
        _body,
        grid=(grid,),
        in_specs=[mk(q) for q in range(NV)] + [mky(q) for q in range(NV)],
        out_specs=pl.BlockSpec((1, 1), lambda i: (0, 0), memory_space=pltpu.SMEM),
        out_shape=jax.ShapeDtypeStruct((1, 1), jnp.float32),
    )(*([y_hat] * NV + [y2] * NV))
    return out[0, 0]


# speedup vs baseline: 2.0057x; 1.0424x over previous
import jax
import jax.numpy as jnp
from jax.experimental import pallas as pl
from jax.experimental.pallas import tpu as pltpu

_SCALE = 30.0
_R2 = 0.7071067811865476
_LOG2E = 1.4426950408889634
_A = _SCALE * _LOG2E
NV = 4
BLK = 512


def _psi(c):
    c = jnp.clip(c, -1.0, 1.0)
    c2 = c * c
    cos4 = 8.0 * c2 * c2 - 8.0 * c2 + 1.0
    k = (
        (c <= _R2).astype(jnp.int32)
        + (c <= 0.0).astype(jnp.int32)
        + (c <= -_R2).astype(jnp.int32)
    )
    co = jnp.where((k & 1) == 1, -1.0, 1.0)
    return co * cos4 - 2.0 * k.astype(jnp.float32)


def _sub_sums(yh, yv):
    cols = jax.lax.broadcasted_iota(jnp.int32, yh.shape, 1)
    mask = cols == yv
    c = jnp.sum(jnp.where(mask, yh, 0.0), axis=1, keepdims=True)
    s0 = jnp.sum(jnp.exp2(yh * _A), axis=1, keepdims=True)
    return c, s0


def _body(*refs):
    out_ref = refs[-1]
    yh_refs = refs[:NV]
    y_refs = refs[NV:2 * NV]
    i = pl.program_id(0)
    nsteps = pl.num_programs(0)

    cs, s0s = [], []
    for q in range(NV):
        cq, s0q = _sub_sums(yh_refs[q][...], y_refs[q][...])
        cs.append(cq)
        s0s.append(s0q)
    c = jnp.concatenate(cs, axis=1)
    s0 = jnp.concatenate(s0s, axis=1)
    psi = _psi(c)
    s = s0 - jnp.exp2(c * _A) + jnp.exp2(psi * _A)
    part = jnp.sum(jnp.log(s) - _SCALE * psi)

    @pl.when(i == 0)
    def _init():
        out_ref[0, 0] = 0.0

    out_ref[0, 0] += part

    @pl.when(i == nsteps - 1)
    def _final():
        out_ref[0, 0] = out_ref[0, 0] * (1.0 / (nsteps * NV * refs[0].shape[0]))


def kernel(y_hat, y):
    n, num_class = y_hat.shape
    blk = BLK
    grid = n // (NV * blk)
    y2 = y.reshape(n, 1)

    def mk(q):
        return pl.BlockSpec((blk, num_class), lambda i, q=q: (NV * i + q, 0))

    def mky(q):
        return pl.BlockSpec((blk, 1), lambda i, q=q: (NV * i + q, 0))

    out = pl.pallas_call(
        _body,
        grid=(grid,),
        in_specs=[mk(q) for q in range(NV)] + [mky(q) for q in range(NV)],
        out_specs=pl.BlockSpec((1, 1), lambda i: (0, 0), memory_space=pltpu.SMEM),
        out_shape=jax.ShapeDtypeStruct((1, 1), jnp.float32),
    )(*([y_hat] * NV + [y2] * NV))
    return out[0, 0]
